# baseline (speedup 1.0000x reference)
import functools

import jax
import jax.numpy as jnp
from jax import lax
from jax.experimental import pallas as pl
from jax.experimental.pallas import tpu as pltpu
from jax.experimental.pallas import tpu_sc as plsc

N = 131072
D = 8
NSEG = 512
NROW = 640
BROW = 544
NW = 32
PTS_W = N // NW
TBL = NROW * 16

NB_PTS = 2048
N_SC = 57344
SC_OFF = N_SC // NB_PTS
TC_BLKS = N // NB_PTS - SC_OFF
PTS_W_SC = N_SC // NW

XC_LO = 1.0000005000002917e-06
XC_HI = 13.815510557964274
LN2 = 0.6931471805599453
_LOGC = (4.342868489e-02, -7.914951135e-02, 1.418487937e-01,
         -3.199195022e-01, 9.618147814e-01, 5.849542865e-01)

_DN2 = (((1,), (0,)), ((), ()))
_PREC = lax.Precision.HIGHEST


@functools.cache
def _get_sc_stats():
    mesh = plsc.VectorSubcoreMesh(core_axis_name="c", subcore_axis_name="s")
    return functools.partial(
        pl.kernel,
        mesh=mesh,
        out_type=jax.ShapeDtypeStruct((NW, TBL), jnp.float32),
        compiler_params=pltpu.CompilerParams(needs_layout_passes=False),
        scratch_types=[
            pltpu.VMEM((D, PTS_W), jnp.float32),
            pltpu.VMEM((PTS_W,), jnp.float32),
            pltpu.VMEM((PTS_W,), jnp.int32),
            pltpu.VMEM((PTS_W,), jnp.int32),
            pltpu.VMEM((PTS_W,), jnp.int32),
            pltpu.VMEM((TBL,), jnp.float32),
        ],
    )(_sc_stats_body)


def _sc_stats_body(et_hbm, mt_hbm, sl_hbm, cl_hbm, bi_hbm, zeros_hbm, out_hbm,
                   e_v, m_v, sl_v, cl_v, bi_v, tbl):
    cid = lax.axis_index("c")
    sub = lax.axis_index("s")
    wid = sub * 2 + cid
    base = wid * PTS_W

    pltpu.sync_copy(et_hbm.at[:, pl.ds(base, PTS_W)], e_v)
    pltpu.sync_copy(mt_hbm.at[pl.ds(base, PTS_W)], m_v)
    pltpu.sync_copy(sl_hbm.at[pl.ds(base, PTS_W)], sl_v)
    pltpu.sync_copy(cl_hbm.at[pl.ds(base, PTS_W)], cl_v)
    pltpu.sync_copy(bi_hbm.at[pl.ds(base, PTS_W)], bi_v)
    pltpu.sync_copy(zeros_hbm, tbl)

    ones = jnp.ones((16,), jnp.float32)

    def body(i, carry):
        off = i * 16
        sl = sl_v[pl.ds(off, 16)]
        cl = cl_v[pl.ds(off, 16)]
        bi = bi_v[pl.ds(off, 16)]
        seg = bi * 128 + sl * 32 + cl
        seg = jnp.where(sl < 4, seg, 560)
        addr = seg * 16
        for q in range(D):
            plsc.addupdate_scatter(tbl, [addr + q], e_v[q, pl.ds(off, 16)])
        plsc.addupdate_scatter(tbl, [addr + 8], m_v[pl.ds(off, 16)])
        plsc.addupdate_scatter(tbl, [addr + 9], ones)
        plsc.addupdate_scatter(tbl, [(bi + BROW) * 16 + 9], ones)
        return carry

    lax.fori_loop(0, PTS_W // 16, body, 0)

    pltpu.sync_copy(tbl, out_hbm.at[wid])


def _finalize_kernel(tables_ref, stats_out, cmat_out, aux_out):
    acc = tables_ref[0]
    for w in range(1, NW):
        acc = acc + tables_ref[w]
    stats_out[...] = acc
    cnt = acc[0:NSEG, 9:10]
    inv_cnt = 1.0 / jnp.maximum(cnt, 1.0)
    cmat = acc[0:NSEG, 0:D] * inv_cnt
    cmat_out[...] = cmat
    sigma = acc[0:NSEG, 8:9] * inv_cnt
    inv_den = 1.0 / (2.0 * sigma * sigma + 1e-8)
    cnorm2 = jnp.sum(cmat * cmat, axis=1, keepdims=True)
    present = (cnt > 0.0).astype(jnp.float32)
    aux_out[...] = jnp.concatenate([inv_den, cnorm2, sigma, present], axis=1)


@functools.cache
def _get_sc_loss():
    mesh = plsc.VectorSubcoreMesh(core_axis_name="c", subcore_axis_name="s")
    return functools.partial(
        pl.kernel,
        mesh=mesh,
        out_type=jax.ShapeDtypeStruct((NW, 32), jnp.float32),
        compiler_params=pltpu.CompilerParams(needs_layout_passes=False),
        scratch_types=[
            pltpu.VMEM((D, PTS_W_SC), jnp.float32),
            pltpu.VMEM((PTS_W_SC,), jnp.float32),
            pltpu.VMEM((PTS_W_SC,), jnp.int32),
            pltpu.VMEM((PTS_W_SC,), jnp.int32),
            pltpu.VMEM((PTS_W_SC,), jnp.int32),
            pltpu.VMEM((NSEG * D,), jnp.float32),
            pltpu.VMEM((NSEG * 4,), jnp.float32),
            pltpu.VMEM((32,), jnp.float32),
        ],
    )(_sc_loss_body)


def _sc_loss_body(et_hbm, mt_hbm, sl_hbm, cl_hbm, bi_hbm, cmat_hbm, aux_hbm,
                  zeros_hbm, out_hbm,
                  e_v, m_v, sl_v, cl_v, bi_v, cent_v, aux_v, ptbl):
    cid = lax.axis_index("c")
    sub = lax.axis_index("s")
    wid = sub * 2 + cid
    base = wid * PTS_W_SC

    pltpu.sync_copy(et_hbm.at[:, pl.ds(base, PTS_W_SC)], e_v)
    pltpu.sync_copy(mt_hbm.at[pl.ds(base, PTS_W_SC)], m_v)
    pltpu.sync_copy(sl_hbm.at[pl.ds(base, PTS_W_SC)], sl_v)
    pltpu.sync_copy(cl_hbm.at[pl.ds(base, PTS_W_SC)], cl_v)
    pltpu.sync_copy(bi_hbm.at[pl.ds(base, PTS_W_SC)], bi_v)
    pltpu.sync_copy(cmat_hbm, cent_v)
    pltpu.sync_copy(aux_hbm, aux_v)
    pltpu.sync_copy(zeros_hbm, ptbl)

    NV = 4

    @plsc.parallel_loop(0, PTS_W_SC // (16 * NV), 1, unroll=1)
    def _loop(i):
        offs = [i * 16 * NV + v * 16 for v in range(NV)]
        sls = [sl_v[pl.ds(o, 16)] for o in offs]
        cls_ = [cl_v[pl.ds(o, 16)] for o in offs]
        bis = [bi_v[pl.ds(o, 16)] for o in offs]
        m16s = [m_v[pl.ds(o, 16)] for o in offs]
        valids = [s < 4 for s in sls]
        gbases = [jnp.where(valids[v], (bis[v] * 4 + sls[v]) * 32, 0)
                  for v in range(NV)]
        eds = [[e_v[d, pl.ds(o, 16)] for d in range(D)] for o in offs]
        en2s = []
        for v in range(NV):
            ed = eds[v]
            s01 = ed[0] * ed[0] + ed[1] * ed[1]
            s23 = ed[2] * ed[2] + ed[3] * ed[3]
            s45 = ed[4] * ed[4] + ed[5] * ed[5]
            s67 = ed[6] * ed[6] + ed[7] * ed[7]
            en2s.append((s01 + s23) + (s45 + s67))

        gids = [jnp.where(valids[v], bis[v] * 4 + sls[v], 0)
                for v in range(NV)]
        accs = [jnp.zeros((16,), jnp.float32) for _ in range(NV)]
        for c in range(32):
            for v in range(NV):
                ed = eds[v]
                idxc = gbases[v] + c
                adr = idxc * D
                g = [plsc.load_gather(cent_v, [adr + d]) for d in range(D)]
                d01 = ed[0] * g[0] + ed[1] * g[1]
                d23 = ed[2] * g[2] + ed[3] * g[3]
                d45 = ed[4] * g[4] + ed[5] * g[5]
                d67 = ed[6] * g[6] + ed[7] * g[7]
                dot = (d01 + d23) + (d45 + d67)
                invd = plsc.load_gather(aux_v, [idxc * 4])
                cn2 = plsc.load_gather(aux_v, [idxc * 4 + 1])
                pres = plsc.load_gather(aux_v, [idxc * 4 + 3])
                x = (cn2 - 2.0 * dot + en2s[v]) * invd
                xc = jnp.clip(x, XC_LO, XC_HI)
                p = jnp.exp(-xc)
                z = 1.0 - p
                bits = plsc.bitcast(z, jnp.int32)
                ebits = lax.shift_right_logical(bits, 23) - 127
                mant = plsc.bitcast(
                    jnp.bitwise_or(jnp.bitwise_and(bits, 0x007FFFFF),
                                   0x3F800000),
                    jnp.float32)
                t = mant - 1.5
                poly = jnp.float32(_LOGC[0])
                for cc in _LOGC[1:]:
                    poly = poly * t + cc
                lnz = (poly + ebits.astype(jnp.float32)) * LN2
                term = jnp.where(cls_[v] == c, xc, -lnz)
                accs[v] = accs[v] + pres * term

        for v in range(NV):
            plsc.addupdate_scatter(ptbl, [gids[v]], accs[v], mask=valids[v])
            sid_own = gbases[v] + cls_[v]
            sig_own = plsc.load_gather(aux_v, [sid_own * 4 + 2])
            dmm = m16s[v] - sig_own
            plsc.addupdate_scatter(ptbl, [gids[v] + 16], dmm * dmm,
                                   mask=valids[v])

    pltpu.sync_copy(ptbl, out_hbm.at[wid])


def _tc_loss_kernel(cmat_ref, aux_ref, et_ref, mt_ref, sl_ref, cl_ref, bi_ref,
                    tc_out):
    step = pl.program_id(0)

    @pl.when(step == 0)
    def _():
        tc_out[...] = jnp.zeros_like(tc_out)

    invd = aux_ref[:, 0:1]
    cn2 = aux_ref[:, 1:2]
    sigma = aux_ref[:, 2:3]

    e = et_ref[...]
    m = mt_ref[...]
    sl = sl_ref[0]
    cl = cl_ref[0]
    bi = bi_ref[0]
    sid = bi * 128 + sl * 32 + cl
    gid = bi * 4 + sl

    enorm2 = jnp.sum(e * e, axis=0, keepdims=True)
    dotp = lax.dot_general(cmat_ref[...], e, _DN2,
                           preferred_element_type=jnp.float32,
                           precision=_PREC)
    x = (cn2 - 2.0 * dotp + enorm2) * invd
    xc = jnp.clip(x, XC_LO, XC_HI)
    p = jnp.exp(-xc)
    log1mp = jnp.log1p(-p)

    rows = lax.broadcasted_iota(jnp.int32, (NSEG, NB_PTS), 0)
    tgt = rows == sid
    g_lo = gid * 32
    valid = (rows >= g_lo) & (rows < g_lo + 32) & (sl < 4)
    term = jnp.where(tgt, xc, -log1mp)
    term = jnp.where(valid, term, 0.0)
    term = term * aux_ref[:, 3:4]
    bce_col = jnp.sum(term, axis=1, keepdims=True)

    dm = m - sigma
    smooth = jnp.where(tgt & valid, dm * dm, 0.0)
    sm_col = jnp.sum(smooth, axis=1, keepdims=True)
    pad = jnp.zeros((NSEG, 14), jnp.float32)
    tc_out[...] += jnp.concatenate([bce_col, sm_col, pad], axis=1)


def _combine_kernel(parts_ref, tc_ref, stats_ref, out_ref):
    acc = parts_ref[0]
    for w in range(1, NW):
        acc = acc + parts_ref[w]
    acc = acc.reshape(1, 32)
    bce_row = acc[:, 0:16]
    sm_row = acc[:, 16:32]

    ii = lax.broadcasted_iota(jnp.int32, (16, 16), 0)
    jj = lax.broadcasted_iota(jnp.int32, (16, 16), 1)
    ident = (ii == jj).astype(jnp.float32)
    ones16 = jnp.ones((16, 1), jnp.float32)

    def tcol(row):
        return lax.dot_general(ident * row, ones16, _DN2,
                               preferred_element_type=jnp.float32,
                               precision=_PREC)

    cnt = stats_ref[0:NSEG, 9:10]

    gi = lax.broadcasted_iota(jnp.int32, (16, NSEG), 0)
    si = lax.broadcasted_iota(jnp.int32, (16, NSEG), 1)
    m1 = ((si // 32) == gi).astype(jnp.float32)
    bi4 = lax.broadcasted_iota(jnp.int32, (4, 16), 0)
    gi16 = lax.broadcasted_iota(jnp.int32, (4, 16), 1)
    m2 = ((gi16 // 4) == bi4).astype(jnp.float32)

    def gdot(mat, vec):
        return lax.dot_general(mat, vec, _DN2,
                               preferred_element_type=jnp.float32,
                               precision=_PREC)

    present = (cnt > 0.0).astype(jnp.float32)
    bce_g = tcol(bce_row) + gdot(m1, tc_ref[:, 0:1])
    sm_g = tcol(sm_row) + gdot(m1, tc_ref[:, 1:2])
    n_sel = gdot(m1, cnt)
    npres = gdot(m1, present)
    n_sel_safe = jnp.maximum(n_sel, 1.0)
    npres_safe = jnp.maximum(npres, 1.0)
    ml = bce_g / n_sel_safe / npres_safe
    sml = sm_g / npres_safe
    s_present = (n_sel > 0.0).astype(jnp.float32)
    contrib = s_present * (ml + sml)
    cls_sum = gdot(m2, contrib)
    cls_cnt = gdot(m2, s_present)
    batch_loss = cls_sum / jnp.maximum(cls_cnt, 1.0)
    bcnt = stats_ref[BROW:BROW + 4, 9:10]
    b_present = (bcnt > 0.0).astype(jnp.float32)
    num = jnp.sum(b_present * batch_loss, keepdims=True)
    den = jnp.maximum(jnp.sum(b_present, keepdims=True), 1.0)
    out_ref[...] = num / den


@jax.jit
def kernel(embeddings, margins, slabels, clabels, batch_idx):
    et = embeddings.T
    mt = margins.reshape(N)
    sl32 = slabels.astype(jnp.int32)
    cl32 = clabels.astype(jnp.int32)
    bi32 = batch_idx.astype(jnp.int32)
    zeros_t = jnp.zeros((TBL,), jnp.float32)
    zeros_p = jnp.zeros((32,), jnp.float32)

    tables = _get_sc_stats()(et, mt, sl32, cl32, bi32, zeros_t)
    tables = tables.reshape(NW, NROW, 16)

    stats, cmat, aux = pl.pallas_call(
        _finalize_kernel,
        out_shape=[jax.ShapeDtypeStruct((NROW, 16), jnp.float32),
                   jax.ShapeDtypeStruct((NSEG, D), jnp.float32),
                   jax.ShapeDtypeStruct((NSEG, 4), jnp.float32)],
    )(tables)

    parts = _get_sc_loss()(et, mt, sl32, cl32, bi32,
                           cmat.reshape(NSEG * D), aux.reshape(NSEG * 4),
                           zeros_p)

    sl = sl32.reshape(N // NB_PTS, 1, NB_PTS)
    cl = cl32.reshape(N // NB_PTS, 1, NB_PTS)
    bi = bi32.reshape(N // NB_PTS, 1, NB_PTS)
    mt2 = mt.reshape(1, N)

    int_spec = pl.BlockSpec((1, 1, NB_PTS), lambda j: (j + SC_OFF, 0, 0))
    et_spec = pl.BlockSpec((D, NB_PTS), lambda j: (0, j + SC_OFF))
    mt_spec = pl.BlockSpec((1, NB_PTS), lambda j: (0, j + SC_OFF))
    full = pl.BlockSpec((NSEG, D), lambda j: (0, 0))
    full4 = pl.BlockSpec((NSEG, 4), lambda j: (0, 0))
    acc_spec = pl.BlockSpec((NSEG, 16), lambda j: (0, 0))

    tc_parts = pl.pallas_call(
        _tc_loss_kernel,
        grid=(TC_BLKS,),
        in_specs=[full, full4, et_spec, mt_spec, int_spec, int_spec, int_spec],
        out_specs=acc_spec,
        out_shape=jax.ShapeDtypeStruct((NSEG, 16), jnp.float32),
    )(cmat, aux, et, mt2, sl, cl, bi)

    out = pl.pallas_call(
        _combine_kernel,
        out_shape=jax.ShapeDtypeStruct((1, 1), jnp.float32),
    )(parts, tc_parts, stats)

    return out[0, 0]


# speedup vs baseline: 1.1945x; 1.1945x over previous
import functools

import jax
import jax.numpy as jnp
from jax import lax
from jax.experimental import pallas as pl
from jax.experimental.pallas import tpu as pltpu
from jax.experimental.pallas import tpu_sc as plsc

N = 131072
D = 8
NSEG = 512
NROW = 640
BROW = 544
NW = 32
PTS_W = N // NW
TBL = NROW * 16

NB_PTS = 2048
N_SC = 40960
SC_OFF = N_SC // NB_PTS
TC_BLKS = N // NB_PTS - SC_OFF
PTS_W_SC = N_SC // NW

XC_LO = 1.0000005000002917e-06
XC_HI = 13.815510557964274
LN2 = 0.6931471805599453
_LOGC = (4.342868489e-02, -7.914951135e-02, 1.418487937e-01,
         -3.199195022e-01, 9.618147814e-01, 5.849542865e-01)

_DN2 = (((1,), (0,)), ((), ()))
_PREC = lax.Precision.HIGHEST


@functools.cache
def _get_sc_stats():
    mesh = plsc.VectorSubcoreMesh(core_axis_name="c", subcore_axis_name="s")
    return functools.partial(
        pl.kernel,
        mesh=mesh,
        out_type=jax.ShapeDtypeStruct((NW, TBL), jnp.float32),
        compiler_params=pltpu.CompilerParams(needs_layout_passes=False),
        scratch_types=[
            pltpu.VMEM((D, PTS_W), jnp.float32),
            pltpu.VMEM((PTS_W,), jnp.float32),
            pltpu.VMEM((PTS_W,), jnp.int32),
            pltpu.VMEM((PTS_W,), jnp.int32),
            pltpu.VMEM((PTS_W,), jnp.int32),
            pltpu.VMEM((TBL,), jnp.float32),
        ],
    )(_sc_stats_body)


def _sc_stats_body(et_hbm, mt_hbm, sl_hbm, cl_hbm, bi_hbm, zeros_hbm, out_hbm,
                   e_v, m_v, sl_v, cl_v, bi_v, tbl):
    cid = lax.axis_index("c")
    sub = lax.axis_index("s")
    wid = sub * 2 + cid
    base = wid * PTS_W

    pltpu.sync_copy(et_hbm.at[:, pl.ds(base, PTS_W)], e_v)
    pltpu.sync_copy(mt_hbm.at[pl.ds(base, PTS_W)], m_v)
    pltpu.sync_copy(sl_hbm.at[pl.ds(base, PTS_W)], sl_v)
    pltpu.sync_copy(cl_hbm.at[pl.ds(base, PTS_W)], cl_v)
    pltpu.sync_copy(bi_hbm.at[pl.ds(base, PTS_W)], bi_v)
    pltpu.sync_copy(zeros_hbm, tbl)

    ones = jnp.ones((16,), jnp.float32)

    def body(i, carry):
        off = i * 16
        sl = sl_v[pl.ds(off, 16)]
        cl = cl_v[pl.ds(off, 16)]
        bi = bi_v[pl.ds(off, 16)]
        seg = bi * 128 + sl * 32 + cl
        seg = jnp.where(sl < 4, seg, 560)
        addr = seg * 16
        for q in range(D):
            plsc.addupdate_scatter(tbl, [addr + q], e_v[q, pl.ds(off, 16)])
        plsc.addupdate_scatter(tbl, [addr + 8], m_v[pl.ds(off, 16)])
        plsc.addupdate_scatter(tbl, [addr + 9], ones)
        plsc.addupdate_scatter(tbl, [(bi + BROW) * 16 + 9], ones)
        return carry

    lax.fori_loop(0, PTS_W // 16, body, 0)

    pltpu.sync_copy(tbl, out_hbm.at[wid])


def _finalize_kernel(tables_ref, stats_out, cmat_out, aux_out):
    acc = tables_ref[0]
    for w in range(1, NW):
        acc = acc + tables_ref[w]
    stats_out[...] = acc
    cnt = acc[0:NSEG, 9:10]
    inv_cnt = 1.0 / jnp.maximum(cnt, 1.0)
    cmat = acc[0:NSEG, 0:D] * inv_cnt
    cmat_out[...] = cmat
    sigma = acc[0:NSEG, 8:9] * inv_cnt
    inv_den = 1.0 / (2.0 * sigma * sigma + 1e-8)
    cnorm2 = jnp.sum(cmat * cmat, axis=1, keepdims=True)
    present = (cnt > 0.0).astype(jnp.float32)
    aux_out[...] = jnp.concatenate([inv_den, cnorm2, sigma, present], axis=1)


@functools.cache
def _get_sc_loss():
    mesh = plsc.VectorSubcoreMesh(core_axis_name="c", subcore_axis_name="s")
    return functools.partial(
        pl.kernel,
        mesh=mesh,
        out_type=jax.ShapeDtypeStruct((NW, 32), jnp.float32),
        compiler_params=pltpu.CompilerParams(needs_layout_passes=False),
        scratch_types=[
            pltpu.VMEM((D, PTS_W_SC), jnp.float32),
            pltpu.VMEM((PTS_W_SC,), jnp.float32),
            pltpu.VMEM((PTS_W_SC,), jnp.int32),
            pltpu.VMEM((PTS_W_SC,), jnp.int32),
            pltpu.VMEM((PTS_W_SC,), jnp.int32),
            pltpu.VMEM((NSEG * D,), jnp.float32),
            pltpu.VMEM((NSEG * 4,), jnp.float32),
            pltpu.VMEM((32,), jnp.float32),
        ],
    )(_sc_loss_body)


def _sc_loss_body(et_hbm, mt_hbm, sl_hbm, cl_hbm, bi_hbm, cmat_hbm, aux_hbm,
                  zeros_hbm, out_hbm,
                  e_v, m_v, sl_v, cl_v, bi_v, cent_v, aux_v, ptbl):
    cid = lax.axis_index("c")
    sub = lax.axis_index("s")
    wid = sub * 2 + cid
    base = wid * PTS_W_SC

    pltpu.sync_copy(et_hbm.at[:, pl.ds(base, PTS_W_SC)], e_v)
    pltpu.sync_copy(mt_hbm.at[pl.ds(base, PTS_W_SC)], m_v)
    pltpu.sync_copy(sl_hbm.at[pl.ds(base, PTS_W_SC)], sl_v)
    pltpu.sync_copy(cl_hbm.at[pl.ds(base, PTS_W_SC)], cl_v)
    pltpu.sync_copy(bi_hbm.at[pl.ds(base, PTS_W_SC)], bi_v)
    pltpu.sync_copy(cmat_hbm, cent_v)
    pltpu.sync_copy(aux_hbm, aux_v)
    pltpu.sync_copy(zeros_hbm, ptbl)

    NV = 4

    @plsc.parallel_loop(0, PTS_W_SC // (16 * NV), 1, unroll=1)
    def _loop(i):
        offs = [i * 16 * NV + v * 16 for v in range(NV)]
        sls = [sl_v[pl.ds(o, 16)] for o in offs]
        cls_ = [cl_v[pl.ds(o, 16)] for o in offs]
        bis = [bi_v[pl.ds(o, 16)] for o in offs]
        m16s = [m_v[pl.ds(o, 16)] for o in offs]
        valids = [s < 4 for s in sls]
        gbases = [jnp.where(valids[v], (bis[v] * 4 + sls[v]) * 32, 0)
                  for v in range(NV)]
        eds = [[e_v[d, pl.ds(o, 16)] for d in range(D)] for o in offs]
        en2s = []
        for v in range(NV):
            ed = eds[v]
            s01 = ed[0] * ed[0] + ed[1] * ed[1]
            s23 = ed[2] * ed[2] + ed[3] * ed[3]
            s45 = ed[4] * ed[4] + ed[5] * ed[5]
            s67 = ed[6] * ed[6] + ed[7] * ed[7]
            en2s.append((s01 + s23) + (s45 + s67))

        gids = [jnp.where(valids[v], bis[v] * 4 + sls[v], 0)
                for v in range(NV)]
        accs = [jnp.zeros((16,), jnp.float32) for _ in range(NV)]
        for c in range(32):
            for v in range(NV):
                ed = eds[v]
                idxc = gbases[v] + c
                adr = idxc * D
                g = [plsc.load_gather(cent_v, [adr + d]) for d in range(D)]
                d01 = ed[0] * g[0] + ed[1] * g[1]
                d23 = ed[2] * g[2] + ed[3] * g[3]
                d45 = ed[4] * g[4] + ed[5] * g[5]
                d67 = ed[6] * g[6] + ed[7] * g[7]
                dot = (d01 + d23) + (d45 + d67)
                invd = plsc.load_gather(aux_v, [idxc * 4])
                cn2 = plsc.load_gather(aux_v, [idxc * 4 + 1])
                pres = plsc.load_gather(aux_v, [idxc * 4 + 3])
                x = (cn2 - 2.0 * dot + en2s[v]) * invd
                xc = jnp.clip(x, XC_LO, XC_HI)
                p = jnp.exp(-xc)
                z = 1.0 - p
                bits = plsc.bitcast(z, jnp.int32)
                ebits = lax.shift_right_logical(bits, 23) - 127
                mant = plsc.bitcast(
                    jnp.bitwise_or(jnp.bitwise_and(bits, 0x007FFFFF),
                                   0x3F800000),
                    jnp.float32)
                t = mant - 1.5
                poly = jnp.float32(_LOGC[0])
                for cc in _LOGC[1:]:
                    poly = poly * t + cc
                lnz = (poly + ebits.astype(jnp.float32)) * LN2
                term = jnp.where(cls_[v] == c, xc, -lnz)
                accs[v] = accs[v] + pres * term

        for v in range(NV):
            plsc.addupdate_scatter(ptbl, [gids[v]], accs[v], mask=valids[v])
            sid_own = gbases[v] + cls_[v]
            sig_own = plsc.load_gather(aux_v, [sid_own * 4 + 2])
            dmm = m16s[v] - sig_own
            plsc.addupdate_scatter(ptbl, [gids[v] + 16], dmm * dmm,
                                   mask=valids[v])

    pltpu.sync_copy(ptbl, out_hbm.at[wid])


def _tc_loss_kernel(cmat_ref, aux_ref, et_ref, mt_ref, sl_ref, cl_ref, bi_ref,
                    tc_out):
    step = pl.program_id(0)

    @pl.when(step == 0)
    def _():
        tc_out[...] = jnp.zeros_like(tc_out)

    invd = aux_ref[:, 0:1]
    cn2 = aux_ref[:, 1:2]
    sigma = aux_ref[:, 2:3]

    e = et_ref[...]
    m = mt_ref[...]
    sl = sl_ref[0]
    cl = cl_ref[0]
    bi = bi_ref[0]
    sid = bi * 128 + sl * 32 + cl
    gid = bi * 4 + sl

    enorm2 = jnp.sum(e * e, axis=0, keepdims=True)
    dotp = lax.dot_general(cmat_ref[...], e, _DN2,
                           preferred_element_type=jnp.float32,
                           precision=lax.Precision.DEFAULT)
    x = (cn2 - 2.0 * dotp + enorm2) * invd
    xc = jnp.clip(x, XC_LO, XC_HI)
    p = jnp.exp(-xc)
    log1mp = jnp.log1p(-p)

    rows = lax.broadcasted_iota(jnp.int32, (NSEG, NB_PTS), 0)
    tgt = rows == sid
    rows_g = lax.broadcasted_iota(jnp.int32, (NSEG, NB_PTS), 0) // 32
    valid = (rows_g == gid) & (sl < 4)
    term = jnp.where(tgt, xc, -log1mp)
    term = jnp.where(valid, term, 0.0)
    term = term * aux_ref[:, 3:4]
    bce_col = jnp.sum(term, axis=1, keepdims=True)

    dm = m - sigma
    smooth = jnp.where(tgt & valid, dm * dm, 0.0)
    sm_col = jnp.sum(smooth, axis=1, keepdims=True)
    pad = jnp.zeros((NSEG, 14), jnp.float32)
    tc_out[...] += jnp.concatenate([bce_col, sm_col, pad], axis=1)


def _combine_kernel(parts_ref, tc_ref, stats_ref, out_ref):
    acc = parts_ref[0]
    for w in range(1, NW):
        acc = acc + parts_ref[w]
    acc = acc.reshape(1, 32)
    bce_row = acc[:, 0:16]
    sm_row = acc[:, 16:32]

    ii = lax.broadcasted_iota(jnp.int32, (16, 16), 0)
    jj = lax.broadcasted_iota(jnp.int32, (16, 16), 1)
    ident = (ii == jj).astype(jnp.float32)
    ones16 = jnp.ones((16, 1), jnp.float32)

    def tcol(row):
        return lax.dot_general(ident * row, ones16, _DN2,
                               preferred_element_type=jnp.float32,
                               precision=_PREC)

    cnt = stats_ref[0:NSEG, 9:10]

    gi = lax.broadcasted_iota(jnp.int32, (16, NSEG), 0)
    si = lax.broadcasted_iota(jnp.int32, (16, NSEG), 1)
    m1 = ((si // 32) == gi).astype(jnp.float32)
    bi4 = lax.broadcasted_iota(jnp.int32, (4, 16), 0)
    gi16 = lax.broadcasted_iota(jnp.int32, (4, 16), 1)
    m2 = ((gi16 // 4) == bi4).astype(jnp.float32)

    def gdot(mat, vec):
        return lax.dot_general(mat, vec, _DN2,
                               preferred_element_type=jnp.float32,
                               precision=_PREC)

    present = (cnt > 0.0).astype(jnp.float32)
    bce_g = tcol(bce_row) + gdot(m1, tc_ref[:, 0:1])
    sm_g = tcol(sm_row) + gdot(m1, tc_ref[:, 1:2])
    n_sel = gdot(m1, cnt)
    npres = gdot(m1, present)
    n_sel_safe = jnp.maximum(n_sel, 1.0)
    npres_safe = jnp.maximum(npres, 1.0)
    ml = bce_g / n_sel_safe / npres_safe
    sml = sm_g / npres_safe
    s_present = (n_sel > 0.0).astype(jnp.float32)
    contrib = s_present * (ml + sml)
    cls_sum = gdot(m2, contrib)
    cls_cnt = gdot(m2, s_present)
    batch_loss = cls_sum / jnp.maximum(cls_cnt, 1.0)
    bcnt = stats_ref[BROW:BROW + 4, 9:10]
    b_present = (bcnt > 0.0).astype(jnp.float32)
    num = jnp.sum(b_present * batch_loss, keepdims=True)
    den = jnp.maximum(jnp.sum(b_present, keepdims=True), 1.0)
    out_ref[...] = num / den


@jax.jit
def kernel(embeddings, margins, slabels, clabels, batch_idx):
    et = embeddings.T
    mt = margins.reshape(N)
    sl32 = slabels.astype(jnp.int32)
    cl32 = clabels.astype(jnp.int32)
    bi32 = batch_idx.astype(jnp.int32)
    zeros_t = jnp.zeros((TBL,), jnp.float32)
    zeros_p = jnp.zeros((32,), jnp.float32)

    tables = _get_sc_stats()(et, mt, sl32, cl32, bi32, zeros_t)
    tables = tables.reshape(NW, NROW, 16)

    stats, cmat, aux = pl.pallas_call(
        _finalize_kernel,
        out_shape=[jax.ShapeDtypeStruct((NROW, 16), jnp.float32),
                   jax.ShapeDtypeStruct((NSEG, D), jnp.float32),
                   jax.ShapeDtypeStruct((NSEG, 4), jnp.float32)],
    )(tables)

    parts = _get_sc_loss()(et, mt, sl32, cl32, bi32,
                           cmat.reshape(NSEG * D), aux.reshape(NSEG * 4),
                           zeros_p)

    sl = sl32.reshape(N // NB_PTS, 1, NB_PTS)
    cl = cl32.reshape(N // NB_PTS, 1, NB_PTS)
    bi = bi32.reshape(N // NB_PTS, 1, NB_PTS)
    mt2 = mt.reshape(1, N)

    int_spec = pl.BlockSpec((1, 1, NB_PTS), lambda j: (j + SC_OFF, 0, 0))
    et_spec = pl.BlockSpec((D, NB_PTS), lambda j: (0, j + SC_OFF))
    mt_spec = pl.BlockSpec((1, NB_PTS), lambda j: (0, j + SC_OFF))
    full = pl.BlockSpec((NSEG, D), lambda j: (0, 0))
    full4 = pl.BlockSpec((NSEG, 4), lambda j: (0, 0))
    acc_spec = pl.BlockSpec((NSEG, 16), lambda j: (0, 0))

    tc_parts = pl.pallas_call(
        _tc_loss_kernel,
        grid=(TC_BLKS,),
        in_specs=[full, full4, et_spec, mt_spec, int_spec, int_spec, int_spec],
        out_specs=acc_spec,
        out_shape=jax.ShapeDtypeStruct((NSEG, 16), jnp.float32),
    )(cmat, aux, et, mt2, sl, cl, bi)

    out = pl.pallas_call(
        _combine_kernel,
        out_shape=jax.ShapeDtypeStruct((1, 1), jnp.float32),
    )(parts, tc_parts, stats)

    return out[0, 0]
